# Initial kernel scaffold; baseline (speedup 1.0000x reference)
import jax
import jax.numpy as jnp
from jax.experimental import pallas as pl


def kernel(inputs):
    raise NotImplementedError("write your pallas kernel here")


# speedup vs baseline: 4.3796x; 4.3796x over previous
import numpy as np

import jax
import jax.numpy as jnp
from jax import lax
from jax.experimental import pallas as pl
from jax.experimental.pallas import tpu as pltpu
from jax.experimental.pallas import tpu_sc as plsc

_S = 768
_GL = 3
_NV = 256
_DV = 8
_NT = _NV // _DV
_ROW = _S * _GL
_CW = _DV * _GL


def _sc_body(a_hbm, ut_hbm, ct_hbm, out_hbm, strip_v, outb_v, ut_v, ct_v):
    wid = lax.axis_index("c") * 16 + lax.axis_index("s")
    pltpu.sync_copy(ut_hbm, ut_v)
    pltpu.sync_copy(ct_hbm, ct_v)

    def strip_loop(t, carry):
        pltpu.sync_copy(a_hbm.at[wid, :, pl.ds(t * _CW, _CW)], strip_v)

        def col_loop(s, inner):
            u_idx = ut_v[pl.ds(s * 16, 16)]
            c_idx = ct_v[pl.ds(s * 16, 16)]
            for v in range(_DV):
                val = plsc.load_gather(strip_v, [u_idx, c_idx + (_GL * v)])
                outb_v[v, pl.ds(s * 16, 16)] = val
            return inner

        lax.fori_loop(0, _ROW // 16, col_loop, 0)
        pltpu.sync_copy(outb_v, out_hbm.at[wid, pl.ds(t * _DV, _DV), :])
        return carry

    lax.fori_loop(0, _NT, strip_loop, 0)


def kernel(inputs):
    A = inputs
    B, H, S1, S2 = A.shape
    a = A.reshape(B * H, S1, S2)
    k = np.arange(_ROW, dtype=np.int32)
    u_tab = jnp.asarray(k // _GL)
    c_tab = jnp.asarray(k % _GL)
    mesh = plsc.VectorSubcoreMesh(
        core_axis_name="c", subcore_axis_name="s", num_cores=2, num_subcores=16
    )
    f = pl.kernel(
        _sc_body,
        mesh=mesh,
        compiler_params=pltpu.CompilerParams(
            use_tc_tiling_on_sc=False, needs_layout_passes=False
        ),
        out_type=jax.ShapeDtypeStruct((B * H, _NV, _ROW), jnp.float32),
        scratch_types=[
            pltpu.VMEM((_S, _CW), jnp.float32),
            pltpu.VMEM((_DV, _ROW), jnp.float32),
            pltpu.VMEM((_ROW,), jnp.int32),
            pltpu.VMEM((_ROW,), jnp.int32),
        ],
    )
    out = f(a, u_tab, c_tab)
    return out.reshape(B, H, S1 * S2 // (_GL * _GL), _GL, _GL)
